# baseline (speedup 1.0000x reference)
import jax
import jax.numpy as jnp
from jax.experimental import pallas as pl
from jax.experimental.pallas import tpu as pltpu

_N = 20000
_ROWS = 160
_COLS = 128
_N_PAD = _ROWS * _COLS
_SROWS = 256
_RM_ROWS = 2
_SCORE_THRESH = 0.05
_NMS_THRESH = 0.5
_K = 100
_NEG = -1e9


def _nms_body(x1_ref, y1_ref, x2_ref, y2_ref, s_in_ref, out_ref, s_ref):
    scores = s_in_ref[...]
    s0 = jnp.where(scores > _SCORE_THRESH, scores, _NEG)
    s_ref[0:_ROWS, :] = s0
    s_ref[_ROWS:_SROWS, :] = jnp.full((_SROWS - _ROWS, _COLS), _NEG, jnp.float32)
    rm0 = jnp.max(s_ref[...].reshape(_RM_ROWS, _COLS, _COLS), axis=2)

    lane = jax.lax.broadcasted_iota(jnp.int32, (1, _COLS), 1)
    rm_r0 = jax.lax.broadcasted_iota(jnp.int32, (_RM_ROWS, _COLS), 0)
    rm_r1 = jax.lax.broadcasted_iota(jnp.int32, (_RM_ROWS, _COLS), 1)
    rm_idx = rm_r0 * _COLS + rm_r1
    big = jnp.int32(2**30)

    def cond(st):
        _, _, _, _, _, _, count, alive = st
        return (count < _K) & alive

    def body(st):
        rm, kx1, ky1, kx2, ky2, karea, count, _ = st
        m = jnp.max(rm)
        alive = m > _NEG / 2.0
        r = jnp.min(jnp.where(rm == m, rm_idx, big))
        r = jnp.where(alive, r, 0)
        row = s_ref[pl.ds(r, 1), :]
        c = jnp.min(jnp.where(row == m, lane, big))
        c = jnp.where(alive, c, 0)
        lmask = lane == c

        def pick(ref):
            return jnp.sum(jnp.where(lmask, ref[pl.ds(r, 1), :], 0.0))

        bx1 = pick(x1_ref)
        by1 = pick(y1_ref)
        bx2 = pick(x2_ref)
        by2 = pick(y2_ref)
        barea = (bx2 - bx1) * (by2 - by1)

        iw = jnp.maximum(jnp.minimum(bx2, kx2) - jnp.maximum(bx1, kx1), 0.0)
        ih = jnp.maximum(jnp.minimum(by2, ky2) - jnp.maximum(by1, ky1), 0.0)
        inter = iw * ih
        rejected = jnp.any(3.0 * inter > karea + (barea + 1e-9))
        accepted = alive & jnp.logical_not(rejected)

        new_row = jnp.where(lmask, _NEG, row)

        @pl.when(alive)
        def _():
            s_ref[pl.ds(r, 1), :] = new_row

        new_rmax = jnp.max(new_row)
        rm = jnp.where(alive & (rm_idx == r), new_rmax, rm)

        smask = lane == count
        kx1 = jnp.where(accepted & smask, bx1, kx1)
        ky1 = jnp.where(accepted & smask, by1, ky1)
        kx2 = jnp.where(accepted & smask, bx2, kx2)
        ky2 = jnp.where(accepted & smask, by2, ky2)
        karea = jnp.where(accepted & smask, barea, karea)

        @pl.when(accepted)
        def _():
            out_ref[count, 0] = bx1
            out_ref[count, 1] = by1
            out_ref[count, 2] = bx2
            out_ref[count, 3] = by2
            out_ref[count, 4] = m

        count = count + jnp.where(accepted, 1, 0)
        return (rm, kx1, ky1, kx2, ky2, karea, count, alive)

    kzero = jnp.zeros((1, _COLS), jnp.float32)
    st = jax.lax.while_loop(
        cond,
        body,
        (rm0, kzero, kzero, kzero, kzero, kzero, jnp.int32(0), jnp.bool_(True)),
    )
    final_count = st[6]

    def zero_fill(i, _):
        out_ref[i, 0] = 0.0
        out_ref[i, 1] = 0.0
        out_ref[i, 2] = 0.0
        out_ref[i, 3] = 0.0
        out_ref[i, 4] = 0.0
        return 0

    jax.lax.fori_loop(final_count, _K, zero_fill, 0)


def kernel(boxes, scores):
    pad = _N_PAD - _N
    x1 = jnp.pad(boxes[:, 0], (0, pad)).reshape(_ROWS, _COLS)
    y1 = jnp.pad(boxes[:, 1], (0, pad)).reshape(_ROWS, _COLS)
    x2 = jnp.pad(boxes[:, 2], (0, pad)).reshape(_ROWS, _COLS)
    y2 = jnp.pad(boxes[:, 3], (0, pad)).reshape(_ROWS, _COLS)
    s = jnp.pad(scores, (0, pad), constant_values=-1.0).reshape(_ROWS, _COLS)

    out = pl.pallas_call(
        _nms_body,
        out_shape=jax.ShapeDtypeStruct((_K, 5), jnp.float32),
        in_specs=[pl.BlockSpec(memory_space=pltpu.VMEM)] * 5,
        out_specs=pl.BlockSpec(memory_space=pltpu.SMEM),
        scratch_shapes=[pltpu.VMEM((_SROWS, _COLS), jnp.float32)],
    )(x1, y1, x2, y2, s)
    return out


# speedup vs baseline: 1.9249x; 1.9249x over previous
import jax
import jax.numpy as jnp
from jax.experimental import pallas as pl
from jax.experimental.pallas import tpu as pltpu

_N = 20000
_ROWS = 160
_COLS = 128
_N_PAD = _ROWS * _COLS
_SCORE_THRESH = 0.05
_NMS_THRESH = 0.5
_K = 100
_NEG = -1e9


def _nms_body(x1_ref, y1_ref, x2_ref, y2_ref, s_ref, out_ref):
    x1 = x1_ref[...]
    y1 = y1_ref[...]
    x2 = x2_ref[...]
    y2 = y2_ref[...]
    scores = s_ref[...]
    s0 = jnp.where(scores > _SCORE_THRESH, scores, _NEG)
    area3 = (x2 - x1) * (y2 - y1) * (1.0 / 3.0)
    rid = jax.lax.broadcasted_iota(jnp.int32, (_ROWS, _COLS), 0)
    cid = jax.lax.broadcasted_iota(jnp.int32, (_ROWS, _COLS), 1)
    idx = rid * _COLS + cid
    lane = jax.lax.broadcasted_iota(jnp.int32, (1, _COLS), 1)

    def _reduce2(x, fn):
        t = fn(x, axis=0, keepdims=True)
        return fn(t, axis=1, keepdims=True)

    def body(i, s):
        m = _reduce2(s, jnp.max)
        cand = jnp.where(s == m, idx, jnp.int32(2**30))
        imin = _reduce2(cand, jnp.min)
        sel = idx == imin
        zero = jnp.zeros_like(s)
        bx1 = _reduce2(jnp.where(sel, x1, zero), jnp.sum)
        by1 = _reduce2(jnp.where(sel, y1, zero), jnp.sum)
        bx2 = _reduce2(jnp.where(sel, x2, zero), jnp.sum)
        by2 = _reduce2(jnp.where(sel, y2, zero), jnp.sum)
        barea3 = (bx2 - bx1) * (by2 - by1) * (1.0 / 3.0)
        valid = m > _NEG / 2.0

        iw = jnp.maximum(jnp.minimum(bx2, x2) - jnp.maximum(bx1, x1), 0.0)
        ih = jnp.maximum(jnp.minimum(by2, y2) - jnp.maximum(by1, y1), 0.0)
        inter = iw * ih
        suppress = inter > area3 + (barea3 + 1e-9 / 3.0)
        s = jnp.where(suppress, _NEG, s)

        row = (
            jnp.where(lane == 0, bx1, 0.0)
            + jnp.where(lane == 1, by1, 0.0)
            + jnp.where(lane == 2, bx2, 0.0)
            + jnp.where(lane == 3, by2, 0.0)
            + jnp.where(lane == 4, m, 0.0)
        )
        out_ref[pl.ds(i, 1), :] = jnp.where(valid, row, 0.0)
        return s

    jax.lax.fori_loop(0, _K, body, s0, unroll=False)


def kernel(boxes, scores):
    pad = _N_PAD - _N
    x1 = jnp.pad(boxes[:, 0], (0, pad)).reshape(_ROWS, _COLS)
    y1 = jnp.pad(boxes[:, 1], (0, pad)).reshape(_ROWS, _COLS)
    x2 = jnp.pad(boxes[:, 2], (0, pad)).reshape(_ROWS, _COLS)
    y2 = jnp.pad(boxes[:, 3], (0, pad)).reshape(_ROWS, _COLS)
    s = jnp.pad(scores, (0, pad), constant_values=-1.0).reshape(_ROWS, _COLS)

    out = pl.pallas_call(
        _nms_body,
        out_shape=jax.ShapeDtypeStruct((_K, _COLS), jnp.float32),
        in_specs=[pl.BlockSpec(memory_space=pltpu.VMEM)] * 5,
        out_specs=pl.BlockSpec(memory_space=pltpu.VMEM),
    )(x1, y1, x2, y2, s)
    return out[:, :5]
